# baseline (speedup 1.0000x reference)
import functools

import jax
import jax.numpy as jnp
import numpy as np
from jax import lax
from jax.experimental import pallas as pl
from jax.experimental.pallas import tpu as pltpu
from jax.experimental.pallas import tpu_sc as plsc

_TAU = 0.5
_K = 10
_EPS = float(np.finfo(np.float32).eps)

_B, _N = 64, 32768
_L = 16
_NSL = _N // _L
_NC, _NS = 2, 16
_NW = _NC * _NS
_RPW = _B // _NW
_PREP_ROWS = 8


_ROT_A = (13, 15, 26, 6)
_ROT_B = (17, 29, 16, 24)


def _np_threefry2x32(k1, k2, x0, x1):
    def rotl(x, d):
        return np.uint32((int(x) << d | int(x) >> (32 - d)) & 0xFFFFFFFF)

    ks = (np.uint32(k1), np.uint32(k2),
          np.uint32(k1) ^ np.uint32(k2) ^ np.uint32(0x1BD11BDA))
    x0 = np.uint32((int(x0) + int(ks[0])) & 0xFFFFFFFF)
    x1 = np.uint32((int(x1) + int(ks[1])) & 0xFFFFFFFF)
    seq = ((_ROT_A, ks[1], ks[2], 1), (_ROT_B, ks[2], ks[0], 2),
           (_ROT_A, ks[0], ks[1], 3), (_ROT_B, ks[1], ks[2], 4),
           (_ROT_A, ks[2], ks[0], 5))
    for rots, ka, kb, c in seq:
        for d in rots:
            x0 = np.uint32((int(x0) + int(x1)) & 0xFFFFFFFF)
            x1 = rotl(x1, d)
            x1 = x0 ^ x1
        x0 = np.uint32((int(x0) + int(ka)) & 0xFFFFFFFF)
        x1 = np.uint32((int(x1) + int(kb) + c) & 0xFFFFFFFF)
    return x0, x1


_FK1, _FK2 = (int(v) for v in _np_threefry2x32(0, 0, 0, 7))


def _uniform_bits(nrows, row0):
    flat = (jnp.arange(row0 * _N, (row0 + nrows) * _N, dtype=jnp.uint32)
            .reshape(nrows, _N))
    ks0 = jnp.uint32(_FK1)
    ks1 = jnp.uint32(_FK2)
    ks2 = jnp.uint32(_FK1 ^ _FK2 ^ 0x1BD11BDA)
    x0 = jnp.full((nrows, _N), ks0, jnp.uint32)
    x1 = flat + ks1
    seq = ((_ROT_A, ks1, ks2, 1), (_ROT_B, ks2, ks0, 2),
           (_ROT_A, ks0, ks1, 3), (_ROT_B, ks1, ks2, 4),
           (_ROT_A, ks2, ks0, 5))
    for rots, ka, kb, cc in seq:
        for d in rots:
            x0 = x0 + x1
            x1 = (x1 << jnp.uint32(d)) | (x1 >> jnp.uint32(32 - d))
            x1 = x0 ^ x1
        x0 = x0 + ka
        x1 = x1 + kb + jnp.uint32(cc)
    bits = x0 ^ x1
    fb = (bits >> jnp.uint32(9)) | jnp.uint32(0x3F800000)
    f = lax.bitcast_convert_type(fb, jnp.float32) - jnp.float32(1.0)
    return jnp.maximum(jnp.float32(_EPS),
                       f * jnp.float32(1.0 - _EPS) + jnp.float32(_EPS))


def _w_init(x, row0, nrows):
    u = _uniform_bits(nrows, row0)
    xs = lax.slice_in_dim(x, row0, row0 + nrows, axis=0)
    g = -jnp.log(-jnp.log(u))
    return jnp.exp((xs + g) * (1.0 / _TAU))


def _sc_iterate(w):
    nrows = w.shape[0]
    rpw = nrows // _NW
    mesh = plsc.VectorSubcoreMesh(core_axis_name="c", subcore_axis_name="s")

    @functools.partial(
        pl.kernel,
        mesh=mesh,
        out_type=jax.ShapeDtypeStruct((nrows, _N), jnp.float32),
        compiler_params=pltpu.CompilerParams(needs_layout_passes=False),
        scratch_types=[
            pltpu.VMEM((_N,), jnp.float32),
            pltpu.VMEM((_N,), jnp.float32),
            pltpu.VMEM((_L,), jnp.float32),
        ],
    )
    def run(w_hbm, out_hbm, wbuf, kbuf, sbuf):
        wid = lax.axis_index("s") * _NC + lax.axis_index("c")
        zeros = jnp.zeros((_L,), jnp.float32)
        lanes = lax.iota(jnp.int32, _L)

        def lane_sum(v):
            for sh in (8, 4, 2, 1):
                sbuf[...] = v
                v = v + plsc.load_gather(sbuf, [lanes ^ sh])
            return v

        for j in range(rpw):
            row = wid * rpw + j
            pltpu.sync_copy(w_hbm.at[row], wbuf)

            def sum_body(off, vsum):
                return vsum + wbuf[pl.ds(off, _L)]

            vsum = plsc.parallel_loop(0, _N, _L, unroll=8,
                                      carry=zeros)(sum_body)
            s = lane_sum(vsum)

            inv0 = 1.0 / s

            def first_body(off, vsum):
                sl = pl.ds(off, _L)
                p = wbuf[sl] * inv0
                kbuf[sl] = p
                mask = jnp.maximum(1.0 - p, _EPS)
                w2 = p * (mask * mask)
                wbuf[sl] = w2
                return vsum + w2

            vsum = plsc.parallel_loop(0, _N, _L, unroll=8,
                                      carry=zeros)(first_body)
            s = lane_sum(vsum)

            def round_body(t, s):
                inv = 1.0 / s

                def body(off, vsum):
                    sl = pl.ds(off, _L)
                    p = wbuf[sl] * inv
                    plsc.addupdate(kbuf.at[sl], p)
                    mask = jnp.maximum(1.0 - p, _EPS)
                    w2 = p * (mask * mask)
                    wbuf[sl] = w2
                    return vsum + w2

                vsum = plsc.parallel_loop(0, _N, _L, unroll=8,
                                          carry=zeros)(body)
                return lane_sum(vsum)

            s = lax.fori_loop(0, _K - 2, round_body, s)

            inv = 1.0 / s

            def last_body(off):
                sl = pl.ds(off, _L)
                plsc.addupdate(kbuf.at[sl], wbuf[sl] * inv)

            plsc.parallel_loop(0, _N, _L, unroll=8)(last_body)
            pltpu.sync_copy(kbuf, out_hbm.at[row])

    return run(w)


def kernel(logits):
    x = jnp.squeeze(logits, 2)
    half = _B // 2
    outs = []
    for h in range(2):
        w = _w_init(x, h * half, half)
        outs.append(_sc_iterate(w))
    return jnp.expand_dims(jnp.concatenate(outs, axis=0), -1)


# speedup vs baseline: 1.0603x; 1.0603x over previous
import functools

import jax
import jax.numpy as jnp
import numpy as np
from jax import lax
from jax.experimental import pallas as pl
from jax.experimental.pallas import tpu as pltpu
from jax.experimental.pallas import tpu_sc as plsc

_TAU = 0.5
_K = 10
_EPS = float(np.finfo(np.float32).eps)

_B, _N = 64, 32768
_L = 16
_NSL = _N // _L
_NC, _NS = 2, 16
_NW = _NC * _NS
_RPW = _B // _NW
_PREP_ROWS = 8


_ROT_A = (13, 15, 26, 6)
_ROT_B = (17, 29, 16, 24)


def _np_threefry2x32(k1, k2, x0, x1):
    def rotl(x, d):
        return np.uint32((int(x) << d | int(x) >> (32 - d)) & 0xFFFFFFFF)

    ks = (np.uint32(k1), np.uint32(k2),
          np.uint32(k1) ^ np.uint32(k2) ^ np.uint32(0x1BD11BDA))
    x0 = np.uint32((int(x0) + int(ks[0])) & 0xFFFFFFFF)
    x1 = np.uint32((int(x1) + int(ks[1])) & 0xFFFFFFFF)
    seq = ((_ROT_A, ks[1], ks[2], 1), (_ROT_B, ks[2], ks[0], 2),
           (_ROT_A, ks[0], ks[1], 3), (_ROT_B, ks[1], ks[2], 4),
           (_ROT_A, ks[2], ks[0], 5))
    for rots, ka, kb, c in seq:
        for d in rots:
            x0 = np.uint32((int(x0) + int(x1)) & 0xFFFFFFFF)
            x1 = rotl(x1, d)
            x1 = x0 ^ x1
        x0 = np.uint32((int(x0) + int(ka)) & 0xFFFFFFFF)
        x1 = np.uint32((int(x1) + int(kb) + c) & 0xFFFFFFFF)
    return x0, x1


_FK1, _FK2 = (int(v) for v in _np_threefry2x32(0, 0, 0, 7))


def _uniform_bits(nrows, row0):
    flat = (jnp.arange(row0 * _N, (row0 + nrows) * _N, dtype=jnp.uint32)
            .reshape(nrows, _N))
    ks0 = jnp.uint32(_FK1)
    ks1 = jnp.uint32(_FK2)
    ks2 = jnp.uint32(_FK1 ^ _FK2 ^ 0x1BD11BDA)
    x0 = jnp.full((nrows, _N), ks0, jnp.uint32)
    x1 = flat + ks1
    seq = ((_ROT_A, ks1, ks2, 1), (_ROT_B, ks2, ks0, 2),
           (_ROT_A, ks0, ks1, 3), (_ROT_B, ks1, ks2, 4),
           (_ROT_A, ks2, ks0, 5))
    for rots, ka, kb, cc in seq:
        for d in rots:
            x0 = x0 + x1
            x1 = (x1 << jnp.uint32(d)) | (x1 >> jnp.uint32(32 - d))
            x1 = x0 ^ x1
        x0 = x0 + ka
        x1 = x1 + kb + jnp.uint32(cc)
    bits = x0 ^ x1
    fb = (bits >> jnp.uint32(9)) | jnp.uint32(0x3F800000)
    f = lax.bitcast_convert_type(fb, jnp.float32) - jnp.float32(1.0)
    return jnp.maximum(jnp.float32(_EPS),
                       f * jnp.float32(1.0 - _EPS) + jnp.float32(_EPS))


def _w_init(x, row0, nrows):
    u = _uniform_bits(nrows, row0)
    xs = lax.slice_in_dim(x, row0, row0 + nrows, axis=0)
    g = -jnp.log(-jnp.log(u))
    return jnp.exp((xs + g) * (1.0 / _TAU))


def _sc_iterate(w):
    nrows = w.shape[0]
    rpw = nrows // _NW
    mesh = plsc.VectorSubcoreMesh(core_axis_name="c", subcore_axis_name="s")

    @functools.partial(
        pl.kernel,
        mesh=mesh,
        out_type=jax.ShapeDtypeStruct((nrows * _N,), jnp.float32),
        compiler_params=pltpu.CompilerParams(needs_layout_passes=False),
        scratch_types=[
            pltpu.VMEM((_N,), jnp.float32),
            pltpu.VMEM((_N,), jnp.float32),
            pltpu.VMEM((_L,), jnp.float32),
        ],
    )
    def run(w_hbm, out_hbm, wbuf, kbuf, sbuf):
        wid = lax.axis_index("s") * _NC + lax.axis_index("c")
        zeros = jnp.zeros((_L,), jnp.float32)
        lanes = lax.iota(jnp.int32, _L)

        def lane_sum(v):
            for sh in (8, 4, 2, 1):
                sbuf[...] = v
                v = v + plsc.load_gather(sbuf, [lanes ^ sh])
            return v

        for j in range(rpw):
            row = wid * rpw + j
            pltpu.sync_copy(w_hbm.at[row], wbuf)

            def sum_body(off, vsum):
                return vsum + wbuf[pl.ds(off, _L)]

            vsum = plsc.parallel_loop(0, _N, _L, unroll=8,
                                      carry=zeros)(sum_body)
            s = lane_sum(vsum)

            inv0 = 1.0 / s

            def first_body(off, vsum):
                sl = pl.ds(off, _L)
                p = wbuf[sl] * inv0
                kbuf[sl] = p
                mask = jnp.maximum(1.0 - p, _EPS)
                w2 = p * (mask * mask)
                wbuf[sl] = w2
                return vsum + w2

            vsum = plsc.parallel_loop(0, _N, _L, unroll=8,
                                      carry=zeros)(first_body)
            s = lane_sum(vsum)

            def round_body(t, s):
                inv = 1.0 / s

                def body(off, vsum):
                    sl = pl.ds(off, _L)
                    p = wbuf[sl] * inv
                    plsc.addupdate(kbuf.at[sl], p)
                    mask = jnp.maximum(1.0 - p, _EPS)
                    w2 = p * (mask * mask)
                    wbuf[sl] = w2
                    return vsum + w2

                vsum = plsc.parallel_loop(0, _N, _L, unroll=8,
                                          carry=zeros)(body)
                return lane_sum(vsum)

            s = lax.fori_loop(0, _K - 2, round_body, s)

            inv = 1.0 / s

            def last_body(off):
                sl = pl.ds(off, _L)
                plsc.addupdate(kbuf.at[sl], wbuf[sl] * inv)

            plsc.parallel_loop(0, _N, _L, unroll=8)(last_body)
            pltpu.sync_copy(kbuf, out_hbm.at[pl.ds(row * _N, _N)])

    return run(w)


def kernel(logits):
    x = jnp.squeeze(logits, 2)
    half = _B // 2
    outs = []
    for h in range(2):
        w = _w_init(x, h * half, half)
        outs.append(_sc_iterate(w))
    return jnp.concatenate(outs, axis=0).reshape(_B, _N, 1)


# speedup vs baseline: 1.1354x; 1.0708x over previous
import functools

import jax
import jax.numpy as jnp
import numpy as np
from jax import lax
from jax.experimental import pallas as pl
from jax.experimental.pallas import tpu as pltpu
from jax.experimental.pallas import tpu_sc as plsc

_TAU = 0.5
_K = 10
_EPS = float(np.finfo(np.float32).eps)

_B, _N = 64, 32768
_L = 16
_NSL = _N // _L
_NC, _NS = 2, 16
_NW = _NC * _NS
_RPW = _B // _NW
_PREP_ROWS = 8


_ROT_A = (13, 15, 26, 6)
_ROT_B = (17, 29, 16, 24)


def _np_threefry2x32(k1, k2, x0, x1):
    def rotl(x, d):
        return np.uint32((int(x) << d | int(x) >> (32 - d)) & 0xFFFFFFFF)

    ks = (np.uint32(k1), np.uint32(k2),
          np.uint32(k1) ^ np.uint32(k2) ^ np.uint32(0x1BD11BDA))
    x0 = np.uint32((int(x0) + int(ks[0])) & 0xFFFFFFFF)
    x1 = np.uint32((int(x1) + int(ks[1])) & 0xFFFFFFFF)
    seq = ((_ROT_A, ks[1], ks[2], 1), (_ROT_B, ks[2], ks[0], 2),
           (_ROT_A, ks[0], ks[1], 3), (_ROT_B, ks[1], ks[2], 4),
           (_ROT_A, ks[2], ks[0], 5))
    for rots, ka, kb, c in seq:
        for d in rots:
            x0 = np.uint32((int(x0) + int(x1)) & 0xFFFFFFFF)
            x1 = rotl(x1, d)
            x1 = x0 ^ x1
        x0 = np.uint32((int(x0) + int(ka)) & 0xFFFFFFFF)
        x1 = np.uint32((int(x1) + int(kb) + c) & 0xFFFFFFFF)
    return x0, x1


_FK1, _FK2 = (int(v) for v in _np_threefry2x32(0, 0, 0, 7))


def _uniform_bits(nrows, row0):
    flat = (jnp.arange(row0 * _N, (row0 + nrows) * _N, dtype=jnp.uint32)
            .reshape(nrows, _N))
    ks0 = jnp.uint32(_FK1)
    ks1 = jnp.uint32(_FK2)
    ks2 = jnp.uint32(_FK1 ^ _FK2 ^ 0x1BD11BDA)
    x0 = jnp.full((nrows, _N), ks0, jnp.uint32)
    x1 = flat + ks1
    seq = ((_ROT_A, ks1, ks2, 1), (_ROT_B, ks2, ks0, 2),
           (_ROT_A, ks0, ks1, 3), (_ROT_B, ks1, ks2, 4),
           (_ROT_A, ks2, ks0, 5))
    for rots, ka, kb, cc in seq:
        for d in rots:
            x0 = x0 + x1
            x1 = (x1 << jnp.uint32(d)) | (x1 >> jnp.uint32(32 - d))
            x1 = x0 ^ x1
        x0 = x0 + ka
        x1 = x1 + kb + jnp.uint32(cc)
    bits = x0 ^ x1
    fb = (bits >> jnp.uint32(9)) | jnp.uint32(0x3F800000)
    f = lax.bitcast_convert_type(fb, jnp.float32) - jnp.float32(1.0)
    return jnp.maximum(jnp.float32(_EPS),
                       f * jnp.float32(1.0 - _EPS) + jnp.float32(_EPS))


def _w_init(x, row0, nrows):
    u = _uniform_bits(nrows, row0)
    xs = lax.slice_in_dim(x, row0, row0 + nrows, axis=0)
    g = -jnp.log(-jnp.log(u))
    return jnp.exp((xs + g) * (1.0 / _TAU))


def _sc_iterate(w, prev=None):
    nrows = w.shape[0]
    rpw = nrows // _NW
    out_elems = (nrows * _N) if prev is None else (2 * nrows * _N)
    mesh = plsc.VectorSubcoreMesh(core_axis_name="c", subcore_axis_name="s")

    scratch = [
        pltpu.VMEM((_N,), jnp.float32),
        pltpu.VMEM((_N,), jnp.float32),
        pltpu.VMEM((_L,), jnp.float32),
    ]
    if prev is not None:
        scratch += [pltpu.VMEM((_N,), jnp.float32),
                    pltpu.SemaphoreType.DMA]

    @functools.partial(
        pl.kernel,
        mesh=mesh,
        out_type=jax.ShapeDtypeStruct((out_elems,), jnp.float32),
        compiler_params=pltpu.CompilerParams(needs_layout_passes=False),
        scratch_types=scratch,
    )
    def run(w_hbm, *rest):
        if prev is None:
            out_hbm, wbuf, kbuf, sbuf = rest
            prev_copy = None
        else:
            prev_hbm, out_hbm, wbuf, kbuf, sbuf, bbuf, sem = rest
        wid = lax.axis_index("s") * _NC + lax.axis_index("c")
        zeros = jnp.zeros((_L,), jnp.float32)
        lanes = lax.iota(jnp.int32, _L)
        out_base = 0 if prev is None else nrows * _N

        def lane_sum(v):
            for sh in (8, 4, 2, 1):
                sbuf[...] = v
                v = v + plsc.load_gather(sbuf, [lanes ^ sh])
            return v

        for j in range(rpw):
            row = wid * rpw + j
            if prev is not None:
                prev_copy = pltpu.async_copy(
                    prev_hbm.at[pl.ds(row * _N, _N)], bbuf, sem)
            pltpu.sync_copy(w_hbm.at[row], wbuf)

            def sum_body(off, vsum):
                return vsum + wbuf[pl.ds(off, _L)]

            vsum = plsc.parallel_loop(0, _N, _L, unroll=8,
                                      carry=zeros)(sum_body)
            s = lane_sum(vsum)

            inv0 = 1.0 / s

            def first_body(off, vsum):
                sl = pl.ds(off, _L)
                p = wbuf[sl] * inv0
                kbuf[sl] = p
                mask = jnp.maximum(1.0 - p, _EPS)
                w2 = p * (mask * mask)
                wbuf[sl] = w2
                return vsum + w2

            vsum = plsc.parallel_loop(0, _N, _L, unroll=8,
                                      carry=zeros)(first_body)
            s = lane_sum(vsum)

            def round_body(t, s):
                inv = 1.0 / s

                def body(off, vsum):
                    sl = pl.ds(off, _L)
                    p = wbuf[sl] * inv
                    plsc.addupdate(kbuf.at[sl], p)
                    mask = jnp.maximum(1.0 - p, _EPS)
                    w2 = p * (mask * mask)
                    wbuf[sl] = w2
                    return vsum + w2

                vsum = plsc.parallel_loop(0, _N, _L, unroll=8,
                                          carry=zeros)(body)
                return lane_sum(vsum)

            s = lax.fori_loop(0, _K - 2, round_body, s)

            inv = 1.0 / s

            def last_body(off):
                sl = pl.ds(off, _L)
                plsc.addupdate(kbuf.at[sl], wbuf[sl] * inv)

            plsc.parallel_loop(0, _N, _L, unroll=8)(last_body)
            pltpu.sync_copy(kbuf,
                            out_hbm.at[pl.ds(out_base + row * _N, _N)])
            if prev is not None:
                prev_copy.wait()
                pltpu.sync_copy(bbuf, out_hbm.at[pl.ds(row * _N, _N)])

    return run(w) if prev is None else run(w, prev)


def kernel(logits):
    x = jnp.squeeze(logits, 2)
    half = _B // 2
    khot0 = _sc_iterate(_w_init(x, 0, half))
    full = _sc_iterate(_w_init(x, half, half), prev=khot0)
    return full.reshape(_B, _N, 1)
